# Initial kernel scaffold; baseline (speedup 1.0000x reference)
import jax
import jax.numpy as jnp
from jax.experimental import pallas as pl


def kernel(positions, amplitudes, hermite_basis):
    raise NotImplementedError("write your pallas kernel here")


# speedup vs baseline: 52.6387x; 52.6387x over previous
import functools

import jax
import jax.numpy as jnp
from jax import lax
from jax.experimental import pallas as pl
from jax.experimental.pallas import tpu as pltpu
from jax.experimental.pallas import tpu_sc as plsc

BATCH = 16384
SEQ = 200
NBASIS = 8
RES = 256

TB = 2048


def _table_body(amp_ref, basis_ref, out_ref):
    out_ref[...] = jnp.dot(
        amp_ref[...], basis_ref[...], preferred_element_type=jnp.float32
    )


def _make_table(amplitudes, hermite_basis):
    return pl.pallas_call(
        _table_body,
        grid=(BATCH // TB,),
        in_specs=[
            pl.BlockSpec((TB, NBASIS), lambda i: (i, 0)),
            pl.BlockSpec((NBASIS, RES), lambda i: (0, 0)),
        ],
        out_specs=pl.BlockSpec((TB, RES), lambda i: (i, 0)),
        out_shape=jax.ShapeDtypeStruct((BATCH, RES), jnp.float32),
    )(amplitudes, hermite_basis)


_INFO = plsc.get_sparse_core_info()
NC = _INFO.num_cores
NS = _INFO.num_subcores
NW = NC * NS
ROWS_PER_W = BATCH // NW
CHUNK = 64
N_CHUNKS = ROWS_PER_W // CHUNK
_OFFS = tuple(16 * j for j in range(SEQ // 16)) + (SEQ - 16,)


def _gather_body(table_hbm, pos_hbm, out_hbm, tab_v, pos_v, res_v):
    wid = lax.axis_index("s") * NC + lax.axis_index("c")
    base = wid * ROWS_PER_W

    def chunk_body(ci, _):
        row0 = base + ci * CHUNK
        pltpu.sync_copy(table_hbm.at[pl.ds(row0, CHUNK)], tab_v)
        pltpu.sync_copy(pos_hbm.at[pl.ds(row0, CHUNK)], pos_v)

        def row_body(r, _):
            rvec = jnp.full((16,), r, jnp.int32)
            for off in _OFFS:
                p = pos_v[r, pl.ds(off, 16)]
                q = (p + 1.0) * 127.5
                idx = jnp.clip(q.astype(jnp.int32), 0, RES - 1)
                res_v[r, pl.ds(off, 16)] = plsc.load_gather(tab_v, [rvec, idx])
            return 0

        lax.fori_loop(0, CHUNK, row_body, 0)
        pltpu.sync_copy(res_v, out_hbm.at[pl.ds(row0, CHUNK)])
        return 0

    lax.fori_loop(0, N_CHUNKS, chunk_body, 0)


_gather_call = functools.partial(
    pl.kernel,
    out_type=jax.ShapeDtypeStruct((BATCH, SEQ), jnp.float32),
    mesh=plsc.VectorSubcoreMesh(core_axis_name="c", subcore_axis_name="s"),
    compiler_params=pltpu.CompilerParams(
        use_tc_tiling_on_sc=False, needs_layout_passes=False
    ),
    scratch_types=[
        pltpu.VMEM((CHUNK, RES), jnp.float32),
        pltpu.VMEM((CHUNK, SEQ), jnp.float32),
        pltpu.VMEM((CHUNK, SEQ), jnp.float32),
    ],
)(_gather_body)


def kernel(positions, amplitudes, hermite_basis):
    table = _make_table(amplitudes, hermite_basis)
    return _gather_call(table, positions)


# speedup vs baseline: 57.3101x; 1.0887x over previous
import functools

import jax
import jax.numpy as jnp
from jax import lax
from jax.experimental import pallas as pl
from jax.experimental.pallas import tpu as pltpu
from jax.experimental.pallas import tpu_sc as plsc

BATCH = 16384
SEQ = 200
NBASIS = 8
RES = 256

TB = 2048


def _table_body(amp_ref, basis_ref, out_ref):
    out_ref[...] = jnp.dot(
        amp_ref[...], basis_ref[...], preferred_element_type=jnp.float32
    )


def _make_table(amplitudes, hermite_basis):
    return pl.pallas_call(
        _table_body,
        grid=(BATCH // TB,),
        in_specs=[
            pl.BlockSpec((TB, NBASIS), lambda i: (i, 0)),
            pl.BlockSpec((NBASIS, RES), lambda i: (0, 0)),
        ],
        out_specs=pl.BlockSpec((TB, RES), lambda i: (i, 0)),
        out_shape=jax.ShapeDtypeStruct((BATCH, RES), jnp.float32),
    )(amplitudes, hermite_basis)


_INFO = plsc.get_sparse_core_info()
NC = _INFO.num_cores
NS = _INFO.num_subcores
NW = NC * NS
ROWS_PER_W = BATCH // NW
CHUNK = 64
N_CHUNKS = ROWS_PER_W // CHUNK
_OFFS = tuple(16 * j for j in range(SEQ // 16)) + (SEQ - 16,)


def _gather_body(
    table_hbm,
    pos_hbm,
    out_hbm,
    tab_v,
    pos_v,
    res_v,
    ld_tab0,
    ld_tab1,
    ld_pos0,
    ld_pos1,
    st0,
    st1,
):
    wid = lax.axis_index("s") * NC + lax.axis_index("c")
    base = wid * ROWS_PER_W
    ld_tab = (ld_tab0, ld_tab1)
    ld_pos = (ld_pos0, ld_pos1)
    st = (st0, st1)

    def load_desc(ci, b):
        row0 = base + ci * CHUNK
        return (
            pltpu.make_async_copy(
                table_hbm.at[pl.ds(row0, CHUNK)], tab_v.at[b], ld_tab[b]
            ),
            pltpu.make_async_copy(
                pos_hbm.at[pl.ds(row0, CHUNK)], pos_v.at[b], ld_pos[b]
            ),
        )

    def store_desc(ci, b):
        row0 = base + ci * CHUNK
        return pltpu.make_async_copy(
            res_v.at[b], out_hbm.at[pl.ds(row0, CHUNK)], st[b]
        )

    for d in load_desc(0, 0):
        d.start()
    for ci in range(N_CHUNKS):
        b = ci % 2
        if ci + 1 < N_CHUNKS:
            for d in load_desc(ci + 1, 1 - b):
                d.start()
        for d in load_desc(ci, b):
            d.wait()
        if ci >= 2:
            store_desc(ci - 2, b).wait()

        def row_body(r, _):
            rvec = jnp.full((16,), r, jnp.int32)
            for off in _OFFS:
                p = pos_v[b, r, pl.ds(off, 16)]
                q = (p + 1.0) * 127.5
                idx = jnp.clip(q.astype(jnp.int32), 0, RES - 1)
                res_v[b, r, pl.ds(off, 16)] = plsc.load_gather(
                    tab_v.at[b], [rvec, idx]
                )
            return 0

        lax.fori_loop(0, CHUNK, row_body, 0)
        store_desc(ci, b).start()
    for ci in (N_CHUNKS - 2, N_CHUNKS - 1):
        store_desc(ci, ci % 2).wait()


_gather_call = functools.partial(
    pl.kernel,
    out_type=jax.ShapeDtypeStruct((BATCH, SEQ), jnp.float32),
    mesh=plsc.VectorSubcoreMesh(core_axis_name="c", subcore_axis_name="s"),
    compiler_params=pltpu.CompilerParams(
        use_tc_tiling_on_sc=False, needs_layout_passes=False
    ),
    scratch_types=[
        pltpu.VMEM((2, CHUNK, RES), jnp.float32),
        pltpu.VMEM((2, CHUNK, SEQ), jnp.float32),
        pltpu.VMEM((2, CHUNK, SEQ), jnp.float32),
        pltpu.SemaphoreType.DMA,
        pltpu.SemaphoreType.DMA,
        pltpu.SemaphoreType.DMA,
        pltpu.SemaphoreType.DMA,
        pltpu.SemaphoreType.DMA,
        pltpu.SemaphoreType.DMA,
    ],
)(_gather_body)


def kernel(positions, amplitudes, hermite_basis):
    table = _make_table(amplitudes, hermite_basis)
    return _gather_call(table, positions)


# speedup vs baseline: 92.7332x; 1.6181x over previous
import functools

import jax
import jax.numpy as jnp
from jax import lax
from jax.experimental import pallas as pl
from jax.experimental.pallas import tpu as pltpu
from jax.experimental.pallas import tpu_sc as plsc

BATCH = 16384
SEQ = 200
NBASIS = 8
RES = 256

TB = 2048


def _table_body(amp_ref, basis_ref, out_ref):
    out_ref[...] = jnp.dot(
        amp_ref[...], basis_ref[...], preferred_element_type=jnp.float32
    )


def _make_table(amplitudes, hermite_basis):
    return pl.pallas_call(
        _table_body,
        grid=(BATCH // TB,),
        in_specs=[
            pl.BlockSpec((TB, NBASIS), lambda i: (i, 0)),
            pl.BlockSpec((NBASIS, RES), lambda i: (0, 0)),
        ],
        out_specs=pl.BlockSpec((TB, RES), lambda i: (i, 0)),
        out_shape=jax.ShapeDtypeStruct((BATCH, RES), jnp.float32),
    )(amplitudes, hermite_basis)


_INFO = plsc.get_sparse_core_info()
NC = _INFO.num_cores
NS = _INFO.num_subcores
NW = NC * NS
ROWS_PER_W = BATCH // NW
CHUNK = 64
N_CHUNKS = ROWS_PER_W // CHUNK
_OFFS = tuple(16 * j for j in range(SEQ // 16)) + (SEQ - 16,)


def _gather_body(
    table_hbm,
    pos_hbm,
    out_hbm,
    tab_v,
    pos_v,
    res_v,
    ld_tab0,
    ld_tab1,
    ld_pos0,
    ld_pos1,
    st0,
    st1,
):
    wid = lax.axis_index("s") * NC + lax.axis_index("c")
    base = wid * ROWS_PER_W
    ld_tab = (ld_tab0, ld_tab1)
    ld_pos = (ld_pos0, ld_pos1)
    st = (st0, st1)

    def load_desc(ci, b):
        row0 = base + ci * CHUNK
        return (
            pltpu.make_async_copy(
                table_hbm.at[pl.ds(row0 * RES, CHUNK * RES)], tab_v.at[b], ld_tab[b]
            ),
            pltpu.make_async_copy(
                pos_hbm.at[pl.ds(row0 * SEQ, CHUNK * SEQ)], pos_v.at[b], ld_pos[b]
            ),
        )

    def store_desc(ci, b):
        row0 = base + ci * CHUNK
        return pltpu.make_async_copy(
            res_v.at[b], out_hbm.at[pl.ds(row0 * SEQ, CHUNK * SEQ)], st[b]
        )

    for d in load_desc(0, 0):
        d.start()
    for ci in range(N_CHUNKS):
        b = ci % 2
        if ci + 1 < N_CHUNKS:
            for d in load_desc(ci + 1, 1 - b):
                d.start()
        for d in load_desc(ci, b):
            d.wait()
        if ci >= 2:
            store_desc(ci - 2, b).wait()

        @plsc.parallel_loop(0, CHUNK, 1, unroll=4)
        def row_body(r):
            rbase = r * SEQ
            tbase = r * RES
            for off in _OFFS:
                p = pos_v[b, pl.ds(rbase + off, 16)]
                q = (p + 1.0) * 127.5
                idx = jnp.clip(q.astype(jnp.int32), 0, RES - 1) + tbase
                res_v[b, pl.ds(rbase + off, 16)] = plsc.load_gather(
                    tab_v.at[b], [idx]
                )

        store_desc(ci, b).start()
    for ci in (N_CHUNKS - 2, N_CHUNKS - 1):
        store_desc(ci, ci % 2).wait()


_gather_call = functools.partial(
    pl.kernel,
    out_type=jax.ShapeDtypeStruct((BATCH * SEQ,), jnp.float32),
    mesh=plsc.VectorSubcoreMesh(core_axis_name="c", subcore_axis_name="s"),
    compiler_params=pltpu.CompilerParams(
        use_tc_tiling_on_sc=False, needs_layout_passes=False
    ),
    scratch_types=[
        pltpu.VMEM((2, CHUNK * RES), jnp.float32),
        pltpu.VMEM((2, CHUNK * SEQ), jnp.float32),
        pltpu.VMEM((2, CHUNK * SEQ), jnp.float32),
        pltpu.SemaphoreType.DMA,
        pltpu.SemaphoreType.DMA,
        pltpu.SemaphoreType.DMA,
        pltpu.SemaphoreType.DMA,
        pltpu.SemaphoreType.DMA,
        pltpu.SemaphoreType.DMA,
    ],
)(_gather_body)


def kernel(positions, amplitudes, hermite_basis):
    table = _make_table(amplitudes, hermite_basis)
    out_flat = _gather_call(table.reshape(-1), positions.reshape(-1))
    return out_flat.reshape(BATCH, SEQ)


# speedup vs baseline: 121.5446x; 1.3107x over previous
import functools

import jax
import jax.numpy as jnp
from jax import lax
from jax.experimental import pallas as pl
from jax.experimental.pallas import tpu as pltpu
from jax.experimental.pallas import tpu_sc as plsc

BATCH = 16384
SEQ = 200
NBASIS = 8
RES = 256
SEQ_R = SEQ - 128

TB = 2048


def _pre_body(amp_ref, basis_ref, pos_ref, idxl_ref, idxr_ref, tabl_ref, tabr_ref):
    p = pos_ref[...]
    idx = jnp.clip(((p + 1.0) * 127.5).astype(jnp.int32), 0, RES - 1)
    idxl_ref[...] = idx[:, :128]
    idxr_ref[...] = jnp.zeros((TB, 128), jnp.int32)
    idxr_ref[:, :SEQ_R] = idx[:, 128:]
    amp = amp_ref[...]
    basis = basis_ref[...]
    tabl_ref[...] = jnp.dot(amp, basis[:, :128], preferred_element_type=jnp.float32)
    tabr_ref[...] = jnp.dot(amp, basis[:, 128:], preferred_element_type=jnp.float32)


def _pre_call(amplitudes, hermite_basis, positions):
    n128 = jax.ShapeDtypeStruct((BATCH, 128), jnp.int32)
    f128 = jax.ShapeDtypeStruct((BATCH, 128), jnp.float32)
    blk = lambda i: (i, 0)
    return pl.pallas_call(
        _pre_body,
        grid=(BATCH // TB,),
        in_specs=[
            pl.BlockSpec((TB, NBASIS), blk),
            pl.BlockSpec((NBASIS, RES), lambda i: (0, 0)),
            pl.BlockSpec((TB, SEQ), blk),
        ],
        out_specs=[
            pl.BlockSpec((TB, 128), blk),
            pl.BlockSpec((TB, 128), blk),
            pl.BlockSpec((TB, 128), blk),
            pl.BlockSpec((TB, 128), blk),
        ],
        out_shape=[n128, n128, f128, f128],
    )(amplitudes, hermite_basis, positions)


def _post_body(outl_ref, outr_ref, out_ref):
    out_ref[:, :128] = outl_ref[...]
    out_ref[:, 128:] = outr_ref[:, :SEQ_R]


def _post_call(outl, outr):
    blk = lambda i: (i, 0)
    return pl.pallas_call(
        _post_body,
        grid=(BATCH // TB,),
        in_specs=[pl.BlockSpec((TB, 128), blk), pl.BlockSpec((TB, 128), blk)],
        out_specs=pl.BlockSpec((TB, SEQ), blk),
        out_shape=jax.ShapeDtypeStruct((BATCH, SEQ), jnp.float32),
    )(outl, outr)


_INFO = plsc.get_sparse_core_info()
NC = _INFO.num_cores
NS = _INFO.num_subcores
NW = NC * NS
ROWS_PER_W = BATCH // NW
CHUNK = 64
N_CHUNKS = ROWS_PER_W // CHUNK
_LOG2_CHUNK = CHUNK.bit_length() - 1
_WINDOWS = 2 * CHUNK * 8


def _gather_body(
    idxl_hbm,
    idxr_hbm,
    tabl_hbm,
    tabr_hbm,
    outl_hbm,
    outr_hbm,
    idx_v,
    tab_v,
    res_v,
    ld0,
    ld1,
    st0,
    st1,
):
    wid = lax.axis_index("s") * NC + lax.axis_index("c")
    base = wid * ROWS_PER_W
    ld = (ld0, ld1)
    st = (st0, st1)

    def load_descs(ci, b):
        row0 = base + ci * CHUNK
        sl = pl.ds(row0, CHUNK)
        return (
            pltpu.make_async_copy(idxl_hbm.at[sl], idx_v.at[b, pl.ds(0, CHUNK)], ld[b]),
            pltpu.make_async_copy(idxr_hbm.at[sl], idx_v.at[b, pl.ds(CHUNK, CHUNK)], ld[b]),
            pltpu.make_async_copy(tabl_hbm.at[sl], tab_v.at[b, pl.ds(0, CHUNK)], ld[b]),
            pltpu.make_async_copy(tabr_hbm.at[sl], tab_v.at[b, pl.ds(CHUNK, CHUNK)], ld[b]),
        )

    def store_descs(ci, b):
        row0 = base + ci * CHUNK
        sl = pl.ds(row0, CHUNK)
        return (
            pltpu.make_async_copy(res_v.at[b, pl.ds(0, CHUNK)], outl_hbm.at[sl], st[b]),
            pltpu.make_async_copy(res_v.at[b, pl.ds(CHUNK, CHUNK)], outr_hbm.at[sl], st[b]),
        )

    for d in load_descs(0, 0):
        d.start()
    for ci in range(N_CHUNKS):
        b = ci % 2
        if ci + 1 < N_CHUNKS:
            for d in load_descs(ci + 1, 1 - b):
                d.start()
        for d in load_descs(ci, b):
            d.wait()
        if ci >= 2:
            for d in store_descs(ci - 2, b):
                d.wait()

        @plsc.parallel_loop(0, _WINDOWS, 1, unroll=8)
        def win_body(w):
            row = w >> 3
            col = (w & 7) * 16
            r = row & (CHUNK - 1)
            iv = idx_v[b, row, pl.ds(col, 16)]
            trow = ((iv >> 7) << _LOG2_CHUNK) + r
            tcol = iv & 127
            res_v[b, row, pl.ds(col, 16)] = plsc.load_gather(
                tab_v.at[b], [trow, tcol]
            )

        for d in store_descs(ci, b):
            d.start()
    for ci in (N_CHUNKS - 2, N_CHUNKS - 1):
        for d in store_descs(ci, ci % 2):
            d.wait()


_gather_call = functools.partial(
    pl.kernel,
    out_type=(
        jax.ShapeDtypeStruct((BATCH, 128), jnp.float32),
        jax.ShapeDtypeStruct((BATCH, 128), jnp.float32),
    ),
    mesh=plsc.VectorSubcoreMesh(core_axis_name="c", subcore_axis_name="s"),
    compiler_params=pltpu.CompilerParams(
        use_tc_tiling_on_sc=False, needs_layout_passes=False
    ),
    scratch_types=[
        pltpu.VMEM((2, 2 * CHUNK, 128), jnp.int32),
        pltpu.VMEM((2, 2 * CHUNK, 128), jnp.float32),
        pltpu.VMEM((2, 2 * CHUNK, 128), jnp.float32),
        pltpu.SemaphoreType.DMA,
        pltpu.SemaphoreType.DMA,
        pltpu.SemaphoreType.DMA,
        pltpu.SemaphoreType.DMA,
    ],
)(_gather_body)


def kernel(positions, amplitudes, hermite_basis):
    idxl, idxr, tabl, tabr = _pre_call(amplitudes, hermite_basis, positions)
    outl, outr = _gather_call(idxl, idxr, tabl, tabr)
    return _post_call(outl, outr)


# speedup vs baseline: 190.4199x; 1.5667x over previous
import functools

import jax
import jax.numpy as jnp
from jax import lax
from jax.experimental import pallas as pl
from jax.experimental.pallas import tpu as pltpu
from jax.experimental.pallas import tpu_sc as plsc

BATCH = 16384
SEQ = 200
NBASIS = 8
RES = 256
SEQ_R = SEQ - 128

TB = 2048


def _pre_body(amp_ref, basis_ref, pos_ref, idxl_ref, idxr_ref, tabl_ref, tabr_ref):
    p = pos_ref[...]
    idx = jnp.clip(((p + 1.0) * 127.5).astype(jnp.int32), 0, RES - 1)
    idxl_ref[...] = idx[:128, :].T
    idxr_ref[:, :SEQ_R] = idx[128:, :].T
    idxr_ref[:, SEQ_R:] = jnp.zeros((TB, 128 - SEQ_R), jnp.int32)
    amp = amp_ref[...]
    basis = basis_ref[...]
    cdims = (((0,), (0,)), ((), ()))
    tabl_ref[...] = lax.dot_general(
        amp, basis[:, :128], cdims, preferred_element_type=jnp.float32
    )
    tabr_ref[...] = lax.dot_general(
        amp, basis[:, 128:], cdims, preferred_element_type=jnp.float32
    )


def _pre_call(amp_t, hermite_basis, pos_t):
    n128 = jax.ShapeDtypeStruct((BATCH, 128), jnp.int32)
    f128 = jax.ShapeDtypeStruct((BATCH, 128), jnp.float32)
    blk = lambda i: (i, 0)
    return pl.pallas_call(
        _pre_body,
        grid=(BATCH // TB,),
        in_specs=[
            pl.BlockSpec((NBASIS, TB), lambda i: (0, i)),
            pl.BlockSpec((NBASIS, RES), lambda i: (0, 0)),
            pl.BlockSpec((SEQ, TB), lambda i: (0, i)),
        ],
        out_specs=[
            pl.BlockSpec((TB, 128), blk),
            pl.BlockSpec((TB, 128), blk),
            pl.BlockSpec((TB, 128), blk),
            pl.BlockSpec((TB, 128), blk),
        ],
        out_shape=[n128, n128, f128, f128],
    )(amp_t, hermite_basis, pos_t)


def _post_body(outl_ref, outr_ref, out_ref):
    out_ref[:128, :] = outl_ref[...].T
    out_ref[128:, :] = outr_ref[...].T[:SEQ_R, :]


def _post_call(outl, outr):
    return pl.pallas_call(
        _post_body,
        grid=(BATCH // TB,),
        in_specs=[
            pl.BlockSpec((TB, 128), lambda i: (i, 0)),
            pl.BlockSpec((TB, 128), lambda i: (i, 0)),
        ],
        out_specs=pl.BlockSpec((SEQ, TB), lambda i: (0, i)),
        out_shape=jax.ShapeDtypeStruct((SEQ, BATCH), jnp.float32),
    )(outl, outr)


_INFO = plsc.get_sparse_core_info()
NC = _INFO.num_cores
NS = _INFO.num_subcores
NW = NC * NS
ROWS_PER_W = BATCH // NW
CHUNK = 64
N_CHUNKS = ROWS_PER_W // CHUNK
_LOG2_CHUNK = CHUNK.bit_length() - 1
_WINDOWS = 2 * CHUNK * 8


def _gather_body(
    idxl_hbm,
    idxr_hbm,
    tabl_hbm,
    tabr_hbm,
    outl_hbm,
    outr_hbm,
    idx_v,
    tab_v,
    res_v,
    ld0,
    ld1,
    st0,
    st1,
):
    wid = lax.axis_index("s") * NC + lax.axis_index("c")
    base = wid * ROWS_PER_W
    ld = (ld0, ld1)
    st = (st0, st1)

    def load_descs(ci, b):
        row0 = base + ci * CHUNK
        sl = pl.ds(row0, CHUNK)
        return (
            pltpu.make_async_copy(idxl_hbm.at[sl], idx_v.at[b, pl.ds(0, CHUNK)], ld[b]),
            pltpu.make_async_copy(idxr_hbm.at[sl], idx_v.at[b, pl.ds(CHUNK, CHUNK)], ld[b]),
            pltpu.make_async_copy(tabl_hbm.at[sl], tab_v.at[b, pl.ds(0, CHUNK)], ld[b]),
            pltpu.make_async_copy(tabr_hbm.at[sl], tab_v.at[b, pl.ds(CHUNK, CHUNK)], ld[b]),
        )

    def store_descs(ci, b):
        row0 = base + ci * CHUNK
        sl = pl.ds(row0, CHUNK)
        return (
            pltpu.make_async_copy(res_v.at[b, pl.ds(0, CHUNK)], outl_hbm.at[sl], st[b]),
            pltpu.make_async_copy(res_v.at[b, pl.ds(CHUNK, CHUNK)], outr_hbm.at[sl], st[b]),
        )

    for d in load_descs(0, 0):
        d.start()
    for ci in range(N_CHUNKS):
        b = ci % 2
        if ci + 1 < N_CHUNKS:
            for d in load_descs(ci + 1, 1 - b):
                d.start()
        for d in load_descs(ci, b):
            d.wait()
        if ci >= 2:
            for d in store_descs(ci - 2, b):
                d.wait()

        @plsc.parallel_loop(0, _WINDOWS, 1, unroll=8)
        def win_body(w):
            row = w >> 3
            col = (w & 7) * 16
            r = row & (CHUNK - 1)
            iv = idx_v[b, row, pl.ds(col, 16)]
            trow = ((iv >> 7) << _LOG2_CHUNK) + r
            tcol = iv & 127
            res_v[b, row, pl.ds(col, 16)] = plsc.load_gather(
                tab_v.at[b], [trow, tcol]
            )

        for d in store_descs(ci, b):
            d.start()
    for ci in (N_CHUNKS - 2, N_CHUNKS - 1):
        for d in store_descs(ci, ci % 2):
            d.wait()


_gather_call = functools.partial(
    pl.kernel,
    out_type=(
        jax.ShapeDtypeStruct((BATCH, 128), jnp.float32),
        jax.ShapeDtypeStruct((BATCH, 128), jnp.float32),
    ),
    mesh=plsc.VectorSubcoreMesh(core_axis_name="c", subcore_axis_name="s"),
    compiler_params=pltpu.CompilerParams(
        use_tc_tiling_on_sc=False, needs_layout_passes=False
    ),
    scratch_types=[
        pltpu.VMEM((2, 2 * CHUNK, 128), jnp.int32),
        pltpu.VMEM((2, 2 * CHUNK, 128), jnp.float32),
        pltpu.VMEM((2, 2 * CHUNK, 128), jnp.float32),
        pltpu.SemaphoreType.DMA,
        pltpu.SemaphoreType.DMA,
        pltpu.SemaphoreType.DMA,
        pltpu.SemaphoreType.DMA,
    ],
)(_gather_body)


def kernel(positions, amplitudes, hermite_basis):
    idxl, idxr, tabl, tabr = _pre_call(
        amplitudes.T, hermite_basis, positions.T
    )
    outl, outr = _gather_call(idxl, idxr, tabl, tabr)
    return _post_call(outl, outr).T
